# baseline (speedup 1.0000x reference)
import functools

import jax
import jax.numpy as jnp
from jax import lax
from jax.experimental import pallas as pl
from jax.experimental.pallas import tpu as pltpu
from jax.experimental.pallas import tpu_sc as plsc

N_CLS = 1000
N_CTX = 4
CTX_DIM = 512
CTX_LEN = 77
SUFFIX_LEN = CTX_LEN - 1 - N_CTX
_LANES = 16
_COLS = CTX_DIM // _LANES

_NC = 2
_NS = 16
_NW = _NC * _NS
_BASE = N_CLS // _NW
_REM = N_CLS % _NW


def _body(prefix_hbm, ctx_hbm, suffix_hbm, out_hbm, sbuf, obuf, cbuf, sem_g, sem_p, sem_o):
    wid = lax.axis_index("s") * _NC + lax.axis_index("c")
    lo = wid * _BASE + jnp.minimum(wid, _REM)
    cnt = _BASE + jnp.where(wid < _REM, 1, 0)

    pltpu.sync_copy(ctx_hbm, cbuf)
    for r in range(N_CTX):
        for c in range(_COLS):
            obuf[1 + r, pl.ds(c * _LANES, _LANES)] = cbuf[r, pl.ds(c * _LANES, _LANES)]

    def body(j, carry):
        i = lo + j
        c_s = pltpu.async_copy(suffix_hbm.at[i], sbuf, sem_g)
        c_p = pltpu.async_copy(prefix_hbm.at[i], obuf.at[pl.ds(0, 1)], sem_p)
        c_s.wait()
        c_p.wait()

        def shift_row(r, carry2):
            for c in range(_COLS):
                obuf[r + (N_CTX + 1), pl.ds(c * _LANES, _LANES)] = sbuf[r, pl.ds(c * _LANES, _LANES)]
            return carry2

        lax.fori_loop(0, SUFFIX_LEN, shift_row, 0)
        c_o = pltpu.async_copy(obuf, out_hbm.at[i], sem_o)
        c_o.wait()
        return carry

    lax.fori_loop(0, cnt, body, 0)


def kernel(prefixs, ctx, suffixs):
    mesh = plsc.VectorSubcoreMesh(core_axis_name="c", subcore_axis_name="s")
    run = pl.kernel(
        _body,
        out_type=jax.ShapeDtypeStruct((N_CLS, CTX_LEN, CTX_DIM), jnp.float32),
        mesh=mesh,
        scratch_types=[
            pltpu.VMEM((SUFFIX_LEN, CTX_DIM), jnp.float32),
            pltpu.VMEM((CTX_LEN, CTX_DIM), jnp.float32),
            pltpu.VMEM((N_CTX, CTX_DIM), jnp.float32),
            pltpu.SemaphoreType.DMA,
            pltpu.SemaphoreType.DMA,
            pltpu.SemaphoreType.DMA,
        ],
    )
    return run(prefixs, ctx, suffixs)


# speedup vs baseline: 1.1681x; 1.1681x over previous
import functools

import jax
import jax.numpy as jnp
from jax import lax
from jax.experimental import pallas as pl
from jax.experimental.pallas import tpu as pltpu
from jax.experimental.pallas import tpu_sc as plsc

N_CLS = 1000
N_CTX = 4
CTX_DIM = 512
CTX_LEN = 77
SUFFIX_LEN = CTX_LEN - 1 - N_CTX
_LANES = 16
_COLS = CTX_DIM // _LANES

_NC = 2
_NS = 16
_NW = _NC * _NS
_BASE = N_CLS // _NW
_REM = N_CLS % _NW
_NBUF = 3
_TRIPS = (_BASE + 1 + _NBUF - 1) // _NBUF


def _body(prefix_hbm, ctx_hbm, suffix_hbm, out_hbm,
          buf0, buf1, buf2, pbuf0, pbuf1, pbuf2, cbuf,
          semg0, semg1, semg2, semp0, semp1, semp2, semo0, semo1, semo2):
    bufs = (buf0, buf1, buf2)
    pbufs = (pbuf0, pbuf1, pbuf2)
    semg = (semg0, semg1, semg2)
    semp = (semp0, semp1, semp2)
    semo = (semo0, semo1, semo2)

    wid = lax.axis_index("s") * _NC + lax.axis_index("c")
    lo = wid * _BASE + jnp.minimum(wid, _REM)
    cnt = _BASE + jnp.where(wid < _REM, 1, 0)

    pltpu.sync_copy(ctx_hbm, cbuf)

    def issue(b, i):
        pltpu.async_copy(suffix_hbm.at[i], bufs[b].at[pl.ds(0, SUFFIX_LEN)], semg[b])
        pltpu.async_copy(prefix_hbm.at[i], pbufs[b], semp[b])

    def wait_gather(b, i):
        pltpu.make_async_copy(suffix_hbm.at[i], bufs[b].at[pl.ds(0, SUFFIX_LEN)], semg[b]).wait()
        pltpu.make_async_copy(prefix_hbm.at[i], pbufs[b], semp[b]).wait()

    def wait_scatter(b, i):
        pltpu.make_async_copy(bufs[b], out_hbm.at[i], semo[b]).wait()

    for b in range(_NBUF):
        issue(b, lo + b)

    def turn(k, carry):
        for s in range(_NBUF):
            j = k * _NBUF + s
            i = lo + j

            def slot():
                wait_gather(s, i)

                def shift_row(t, c2):
                    r = (SUFFIX_LEN - 1) - t
                    for c in range(_COLS):
                        bufs[s][r + (N_CTX + 1), pl.ds(c * _LANES, _LANES)] = \
                            bufs[s][r, pl.ds(c * _LANES, _LANES)]
                    return c2

                lax.fori_loop(0, SUFFIX_LEN, shift_row, 0)

                for c in range(_COLS):
                    bufs[s][0, pl.ds(c * _LANES, _LANES)] = \
                        pbufs[s][0, pl.ds(c * _LANES, _LANES)]
                for r in range(N_CTX):
                    for c in range(_COLS):
                        bufs[s][1 + r, pl.ds(c * _LANES, _LANES)] = \
                            cbuf[r, pl.ds(c * _LANES, _LANES)]

                pltpu.async_copy(bufs[s], out_hbm.at[i], semo[s])

                def refill():
                    wait_scatter(s, i)
                    issue(s, i + _NBUF)

                jax.lax.cond(j + _NBUF < cnt, refill, lambda: None)

            if s == 0:
                slot()
            else:
                jax.lax.cond(j < cnt, slot, lambda: None)
        return carry

    lax.fori_loop(0, _TRIPS, turn, 0)

    for b in range(_NBUF):
        wait_scatter(b, lo)


def kernel(prefixs, ctx, suffixs):
    mesh = plsc.VectorSubcoreMesh(core_axis_name="c", subcore_axis_name="s")
    run = pl.kernel(
        _body,
        out_type=jax.ShapeDtypeStruct((N_CLS, CTX_LEN, CTX_DIM), jnp.float32),
        mesh=mesh,
        scratch_types=[
            pltpu.VMEM((CTX_LEN, CTX_DIM), jnp.float32),
            pltpu.VMEM((CTX_LEN, CTX_DIM), jnp.float32),
            pltpu.VMEM((CTX_LEN, CTX_DIM), jnp.float32),
            pltpu.VMEM((1, CTX_DIM), jnp.float32),
            pltpu.VMEM((1, CTX_DIM), jnp.float32),
            pltpu.VMEM((1, CTX_DIM), jnp.float32),
            pltpu.VMEM((N_CTX, CTX_DIM), jnp.float32),
            pltpu.SemaphoreType.DMA,
            pltpu.SemaphoreType.DMA,
            pltpu.SemaphoreType.DMA,
            pltpu.SemaphoreType.DMA,
            pltpu.SemaphoreType.DMA,
            pltpu.SemaphoreType.DMA,
            pltpu.SemaphoreType.DMA,
            pltpu.SemaphoreType.DMA,
            pltpu.SemaphoreType.DMA,
        ],
    )
    return run(prefixs, ctx, suffixs)


# speedup vs baseline: 1.9350x; 1.6565x over previous
import functools

import jax
import jax.numpy as jnp
from jax import lax
from jax.experimental import pallas as pl
from jax.experimental.pallas import tpu as pltpu
from jax.experimental.pallas import tpu_sc as plsc

N_CLS = 1000
N_CTX = 4
CTX_DIM = 512
CTX_LEN = 77
SUFFIX_LEN = CTX_LEN - 1 - N_CTX
_LANES = 16
_COLS = CTX_DIM // _LANES

_NC = 2
_NS = 16
_NW = _NC * _NS
_BASE = N_CLS // _NW
_REM = N_CLS % _NW
_NBUF = 3
_TRIPS = (_BASE + 1 + _NBUF - 1) // _NBUF


def _body(prefix_hbm, ctx_hbm, suffix_hbm, out_hbm,
          buf0, buf1, buf2, pbuf0, pbuf1, pbuf2, cbuf,
          semg0, semg1, semg2, semp0, semp1, semp2, semo0, semo1, semo2):
    bufs = (buf0, buf1, buf2)
    pbufs = (pbuf0, pbuf1, pbuf2)
    semg = (semg0, semg1, semg2)
    semp = (semp0, semp1, semp2)
    semo = (semo0, semo1, semo2)

    wid = lax.axis_index("s") * _NC + lax.axis_index("c")
    lo = wid * _BASE + jnp.minimum(wid, _REM)
    cnt = _BASE + jnp.where(wid < _REM, 1, 0)

    pltpu.sync_copy(ctx_hbm, cbuf)

    def issue(b, i):
        pltpu.async_copy(suffix_hbm.at[i], bufs[b].at[pl.ds(0, SUFFIX_LEN)], semg[b])
        pltpu.async_copy(prefix_hbm.at[i], pbufs[b], semp[b])

    def wait_gather(b, i):
        pltpu.make_async_copy(suffix_hbm.at[i], bufs[b].at[pl.ds(0, SUFFIX_LEN)], semg[b]).wait()
        pltpu.make_async_copy(prefix_hbm.at[i], pbufs[b], semp[b]).wait()

    def wait_scatter(b, i):
        pltpu.make_async_copy(bufs[b], out_hbm.at[i], semo[b]).wait()

    for b in range(_NBUF):
        issue(b, lo + b)

    def turn(k, carry):
        for s in range(_NBUF):
            j = k * _NBUF + s
            i = lo + j

            def slot():
                wait_gather(s, i)

                def shift_row(t, c2):
                    r = (SUFFIX_LEN - 1) - t
                    vals = [bufs[s][r, pl.ds(c * _LANES, _LANES)]
                            for c in range(_COLS)]
                    for c in range(_COLS):
                        bufs[s][r + (N_CTX + 1), pl.ds(c * _LANES, _LANES)] = vals[c]
                    return c2

                lax.fori_loop(0, SUFFIX_LEN, shift_row, 0)

                pvals = [pbufs[s][0, pl.ds(c * _LANES, _LANES)]
                         for c in range(_COLS)]
                for c in range(_COLS):
                    bufs[s][0, pl.ds(c * _LANES, _LANES)] = pvals[c]
                for r in range(N_CTX):
                    cvals = [cbuf[r, pl.ds(c * _LANES, _LANES)]
                             for c in range(_COLS)]
                    for c in range(_COLS):
                        bufs[s][1 + r, pl.ds(c * _LANES, _LANES)] = cvals[c]

                pltpu.async_copy(bufs[s], out_hbm.at[i], semo[s])

                def refill():
                    wait_scatter(s, i)
                    issue(s, i + _NBUF)

                jax.lax.cond(j + _NBUF < cnt, refill, lambda: None)

            if s == 0:
                slot()
            else:
                jax.lax.cond(j < cnt, slot, lambda: None)
        return carry

    lax.fori_loop(0, _TRIPS, turn, 0)

    for b in range(_NBUF):
        wait_scatter(b, lo)


def kernel(prefixs, ctx, suffixs):
    mesh = plsc.VectorSubcoreMesh(core_axis_name="c", subcore_axis_name="s")
    run = pl.kernel(
        _body,
        out_type=jax.ShapeDtypeStruct((N_CLS, CTX_LEN, CTX_DIM), jnp.float32),
        mesh=mesh,
        scratch_types=[
            pltpu.VMEM((CTX_LEN, CTX_DIM), jnp.float32),
            pltpu.VMEM((CTX_LEN, CTX_DIM), jnp.float32),
            pltpu.VMEM((CTX_LEN, CTX_DIM), jnp.float32),
            pltpu.VMEM((1, CTX_DIM), jnp.float32),
            pltpu.VMEM((1, CTX_DIM), jnp.float32),
            pltpu.VMEM((1, CTX_DIM), jnp.float32),
            pltpu.VMEM((N_CTX, CTX_DIM), jnp.float32),
            pltpu.SemaphoreType.DMA,
            pltpu.SemaphoreType.DMA,
            pltpu.SemaphoreType.DMA,
            pltpu.SemaphoreType.DMA,
            pltpu.SemaphoreType.DMA,
            pltpu.SemaphoreType.DMA,
            pltpu.SemaphoreType.DMA,
            pltpu.SemaphoreType.DMA,
            pltpu.SemaphoreType.DMA,
        ],
    )
    return run(prefixs, ctx, suffixs)
